# baseline (speedup 1.0000x reference)
import jax
import jax.numpy as jnp
from jax.experimental import pallas as pl

B, N, M, D = 8, 2048, 2048, 3


def _chamfer_body(t_ref, s_ref, o1_ref, o2_ref):
    t = t_ref[0]
    s = s_ref[0] * -2.0
    tn = jnp.sum(t * t, axis=1, keepdims=True)
    sn = 0.25 * jnp.sum(s * s, axis=1, keepdims=True)
    sn_row = jnp.transpose(sn, (1, 0))
    t_hi = t.astype(jnp.bfloat16)
    t_lo = (t - t_hi.astype(jnp.float32)).astype(jnp.bfloat16)
    s_hi = s.astype(jnp.bfloat16)
    s_lo = (s - s_hi.astype(jnp.float32)).astype(jnp.bfloat16)
    lhs = jnp.concatenate([t_hi, t_lo, t_hi], axis=1)
    rhs = jnp.concatenate([s_hi, s_hi, s_lo], axis=1)
    prod = jax.lax.dot_general(
        lhs, rhs, (((1,), (1,)), ((), ())),
        preferred_element_type=jnp.float32)
    d = prod + tn + sn_row
    rowmin = jnp.maximum(jnp.min(d, axis=1), 0.0)
    colmin = jnp.maximum(jnp.min(d, axis=0), 0.0)
    s1 = jnp.sum(jnp.sqrt(rowmin))
    s2 = jnp.sum(jnp.sqrt(colmin))
    o1_ref[...] = jnp.full((1, 1, 128), s1, dtype=jnp.float32)
    o2_ref[...] = jnp.full((1, 1, 128), s2, dtype=jnp.float32)


def kernel(template, source):
    o1, o2 = pl.pallas_call(
        _chamfer_body,
        grid=(B,),
        in_specs=[
            pl.BlockSpec((1, N, D), lambda b: (b, 0, 0)),
            pl.BlockSpec((1, M, D), lambda b: (b, 0, 0)),
        ],
        out_specs=[
            pl.BlockSpec((1, 1, 128), lambda b: (b, 0, 0)),
            pl.BlockSpec((1, 1, 128), lambda b: (b, 0, 0)),
        ],
        out_shape=[
            jax.ShapeDtypeStruct((B, 1, 128), jnp.float32),
            jax.ShapeDtypeStruct((B, 1, 128), jnp.float32),
        ],
    )(template, source)
    cost_p0_p1 = jnp.sum(o1[:, 0, 0]) / (B * N)
    cost_p1_p0 = jnp.sum(o2[:, 0, 0]) / (B * M)
    return (cost_p0_p1 + cost_p1_p0) / 2.0


# speedup vs baseline: 1.2942x; 1.2942x over previous
import jax
import jax.numpy as jnp
from jax.experimental import pallas as pl
from jax.experimental.pallas import tpu as pltpu

B, N, M, D = 8, 2048, 2048, 3


def _chamfer_body(t_ref, s_ref, o_ref, acc_ref):
    b = pl.program_id(0)
    t = t_ref[0]
    sT = jnp.transpose(s_ref[0], (1, 0)) * -2.0
    tn = jnp.sum(t * t, axis=1, keepdims=True)
    sn = 0.25 * jnp.sum(sT * sT, axis=0, keepdims=True)
    prod = jax.lax.dot_general(
        t, sT, (((1,), (0,)), ((), ())),
        preferred_element_type=jnp.float32)
    d = prod + tn + sn
    rowmin = jnp.maximum(jnp.min(d, axis=1), 0.0)
    colmin = jnp.maximum(jnp.min(d, axis=0), 0.0)
    s1 = jnp.sum(jnp.sqrt(rowmin))
    s2 = jnp.sum(jnp.sqrt(colmin))

    @pl.when(b == 0)
    def _init():
        acc_ref[0] = 0.0
        acc_ref[1] = 0.0

    acc_ref[0] += s1
    acc_ref[1] += s2

    @pl.when(b == B - 1)
    def _fin():
        c1 = acc_ref[0] / (B * N)
        c2 = acc_ref[1] / (B * M)
        o_ref[0, 0] = (c1 + c2) * 0.5


def kernel(template, source):
    out = pl.pallas_call(
        _chamfer_body,
        grid=(B,),
        in_specs=[
            pl.BlockSpec((1, N, D), lambda b: (b, 0, 0)),
            pl.BlockSpec((1, M, D), lambda b: (b, 0, 0)),
        ],
        out_specs=pl.BlockSpec(memory_space=pltpu.SMEM),
        out_shape=jax.ShapeDtypeStruct((1, 1), jnp.float32),
        scratch_shapes=[pltpu.SMEM((2,), jnp.float32)],
    )(template, source)
    return out[0, 0]


# speedup vs baseline: 1.5465x; 1.1950x over previous
import jax
import jax.numpy as jnp
from jax.experimental import pallas as pl
from jax.experimental.pallas import tpu as pltpu

B, N, M, D = 8, 2048, 2048, 3


def _chamfer_body(t_ref, s_ref, o_ref, acc_ref):
    b = pl.program_id(0)
    t = t_ref[0]
    sT = s_ref[0]
    tn = jnp.sum(t * t, axis=1, keepdims=True)
    sn = 0.25 * jnp.sum(sT * sT, axis=0, keepdims=True)
    prod = jax.lax.dot_general(
        t, sT, (((1,), (0,)), ((), ())),
        preferred_element_type=jnp.float32)
    d = prod + tn + sn
    rowmin = jnp.maximum(jnp.min(d, axis=1), 0.0)
    colmin = jnp.maximum(jnp.min(d, axis=0), 0.0)
    s1 = jnp.sum(jnp.sqrt(rowmin))
    s2 = jnp.sum(jnp.sqrt(colmin))

    @pl.when(b == 0)
    def _init():
        acc_ref[0] = 0.0
        acc_ref[1] = 0.0

    acc_ref[0] += s1
    acc_ref[1] += s2

    @pl.when(b == B - 1)
    def _fin():
        c1 = acc_ref[0] / (B * N)
        c2 = acc_ref[1] / (B * M)
        o_ref[0, 0] = (c1 + c2) * 0.5


def kernel(template, source):
    sT = jnp.swapaxes(source, 1, 2) * -2.0
    out = pl.pallas_call(
        _chamfer_body,
        grid=(B,),
        in_specs=[
            pl.BlockSpec((1, N, D), lambda b: (b, 0, 0)),
            pl.BlockSpec((1, D, M), lambda b: (b, 0, 0)),
        ],
        out_specs=pl.BlockSpec(memory_space=pltpu.SMEM),
        out_shape=jax.ShapeDtypeStruct((1, 1), jnp.float32),
        scratch_shapes=[pltpu.SMEM((2,), jnp.float32)],
    )(template, sT)
    return out[0, 0]
